# baseline (speedup 1.0000x reference)
import functools

import jax
import jax.numpy as jnp
from jax import lax
from jax.experimental import pallas as pl
from jax.experimental.pallas import tpu as pltpu
from jax.experimental.pallas import tpu_sc as plsc

N = 320000
D = 128
BLK = 2000

SPROWS = 6144
NCHUNK = 54
STAGE = SPROWS + 16
W = 128
CPC = NCHUNK // 2
TS = SPROWS // 16


def _head_body(c_ref, f_ref, w_ref, b_ref, off_ref, nc_ref, h_ref):
    f = f_ref[...]
    w = w_ref[...]
    off = jnp.dot(f, w, preferred_element_type=jnp.float32) + b_ref[0, :]
    off_ref[...] = off
    ci = c_ref[...]
    oi = off.astype(jnp.int32)
    nc = ci + jnp.concatenate(
        [jnp.zeros((ci.shape[0], 1), jnp.int32), oi], axis=1)
    nc_ref[...] = nc
    c = nc + 1024
    h = ((c[:, 0] * 4096 + c[:, 1]) * 4096 + c[:, 2]) * 4096 + c[:, 3]
    h_ref[...] = h.reshape(1, 1, BLK)


def _head(F, C, W, b):
    nb = N // BLK
    grid = (nb,)
    out_shapes = (
        jax.ShapeDtypeStruct((N, 3), jnp.float32),
        jax.ShapeDtypeStruct((N, 4), jnp.int32),
        jax.ShapeDtypeStruct((nb, 1, BLK), jnp.int32),
    )
    off, nc, h = pl.pallas_call(
        _head_body,
        grid=grid,
        in_specs=[
            pl.BlockSpec((BLK, 4), lambda i: (i, 0)),
            pl.BlockSpec((BLK, D), lambda i: (i, 0)),
            pl.BlockSpec((D, 3), lambda i: (0, 0)),
            pl.BlockSpec((1, 3), lambda i: (0, 0)),
        ],
        out_specs=(
            pl.BlockSpec((BLK, 3), lambda i: (i, 0)),
            pl.BlockSpec((BLK, 4), lambda i: (i, 0)),
            pl.BlockSpec((1, 1, BLK), lambda i: (i, 0, 0)),
        ),
        out_shape=out_shapes,
    )(C, F, W, b.reshape(1, 3))
    return off, nc, h.reshape(-1)


RBLK = 8000

def _rank_body(s_ref, r_ref, prev_ref, cum_ref):
    i = pl.program_id(0)

    @pl.when(i == 0)
    def _init():
        prev_ref[0] = s_ref[0, 0, 0] + 1
        cum_ref[0] = 0

    s = s_ref[0, 0, :]
    s_shift = jnp.concatenate([jnp.full((1,), prev_ref[0], jnp.int32), s[:-1]])
    flag = (s != s_shift).astype(jnp.int32)
    x = flag
    d = 1
    while d < RBLK:
        x = x + jnp.concatenate([jnp.zeros((d,), jnp.int32), x[:-d]])
        d *= 2
    r = x + cum_ref[0] - 1
    r_ref[0, 0, :] = r
    cum_ref[0] = r[-1] + 1
    prev_ref[0] = s[-1]


def _ranks(S):
    nb = N // RBLK
    r = pl.pallas_call(
        _rank_body,
        grid=(nb,),
        in_specs=[pl.BlockSpec((1, 1, RBLK), lambda i: (i, 0, 0))],
        out_specs=pl.BlockSpec((1, 1, RBLK), lambda i: (i, 0, 0)),
        out_shape=jax.ShapeDtypeStruct((nb, 1, RBLK), jnp.int32),
        scratch_shapes=[pltpu.SMEM((1,), jnp.int32), pltpu.SMEM((1,), jnp.int32)],
    )(S.reshape(nb, 1, RBLK))
    return r.reshape(-1)


def _pool_body(f_hbm, p_hbm, r_hbm, blo_hbm, bhi_hbm, z128_hbm,
               feats_out,
               blo_v, bhi_v, idx_v, pidx_v, rv_v, seg_v,
               rows_v, feats_st, sem1):
    core = lax.axis_index("c")
    sub = lax.axis_index("s")
    iota = lax.iota(jnp.int32, 16)

    pltpu.sync_copy(blo_hbm, blo_v)
    pltpu.sync_copy(bhi_hbm, bhi_v)
    zb0 = sub * TS
    pltpu.sync_copy(z128_hbm.at[pl.ds(zb0, TS)], feats_st.at[pl.ds(zb0, TS)])
    plsc.subcore_barrier()

    @pl.loop(0, CPC)
    def chunk_body(i):
        c = 2 * i + core
        lo_c = blo_v[c, :][0]
        hi_c = bhi_v[c, :][0]
        base_seg = c * SPROWS
        length = hi_c - lo_c
        lo_t = lo_c + (((length * sub) // 16) & ~7)
        hi_t = lo_c + (((length * (sub + 1)) // 16) & ~7)
        zbase = sub * TS

        nw = (hi_t - lo_t + (W - 1)) // W

        @pl.loop(0, nw)
        def win_body(w):
            j0 = pl.multiple_of(lo_t + w * W, 8)
            pltpu.sync_copy(p_hbm.at[pl.ds(j0, W)], idx_v)
            pltpu.sync_copy(r_hbm.at[pl.ds(j0, W)], rv_v)
            for k in range(W // 16):
                jvec = j0 + k * 16 + iota
                rv = rv_v[pl.ds(k * 16, 16)]
                seg = rv - base_seg
                ok = (jvec < hi_t) & (seg >= 0) & (seg < SPROWS)
                seg_v[pl.ds(k * 16, 16)] = jnp.where(ok, seg, SPROWS + iota)
                pv = idx_v[pl.ds(k * 16, 16)]
                pidx_v[pl.ds(k * 16, 16)] = jnp.minimum(pv, N - 1)
            pltpu.async_copy(f_hbm.at[pidx_v], rows_v, sem1).wait()
            pltpu.sync_copy(rows_v, feats_st.at[seg_v], add=True)

        plsc.subcore_barrier()

        obase = base_seg + zbase
        pltpu.sync_copy(feats_st.at[pl.ds(zbase, TS)],
                        feats_out.at[pl.ds(obase, TS)])
        pltpu.sync_copy(z128_hbm.at[pl.ds(zbase, TS)], feats_st.at[pl.ds(zbase, TS)])
        plsc.subcore_barrier()


def _pool_sc(F, P_pad, r_pad, blo, bhi):
    mesh = plsc.VectorSubcoreMesh(core_axis_name="c", subcore_axis_name="s")
    pool = pl.kernel(
        _pool_body,
        mesh=mesh,
        out_type=[
            jax.ShapeDtypeStruct((NCHUNK * SPROWS, D), jnp.float32),
        ],
        scratch_types=[
            pltpu.VMEM((160, 16), jnp.int32),
            pltpu.VMEM((160, 16), jnp.int32),
            pltpu.VMEM((W,), jnp.int32),
            pltpu.VMEM((W,), jnp.int32),
            pltpu.VMEM((W,), jnp.int32),
            pltpu.VMEM((W,), jnp.int32),
            pltpu.VMEM((W, D), jnp.float32),
            pltpu.VMEM_SHARED((STAGE, D), jnp.float32),
            pltpu.SemaphoreType.DMA,
        ],
    )
    z128 = jnp.zeros((SPROWS, D), jnp.float32)
    return pool(F, P_pad, r_pad, blo, bhi, z128)[0]


def _finish_body(fs_ref, c0_ref, c1_ref, c2_ref, c3_ref, cnt_ref,
                 feats_ref, coords_ref):
    inv_c = 1.0 / jnp.maximum(cnt_ref[...], 1.0)
    feats_ref[...] = fs_ref[...] * inv_c
    cs = jnp.concatenate(
        [c0_ref[...], c1_ref[...], c2_ref[...], c3_ref[...]], axis=1)
    coords_ref[...] = (cs * inv_c).astype(jnp.int32)


def _finish(feats_sum, cols):
    nb = N // BLK
    col_spec = pl.BlockSpec((BLK, 1), lambda i: (i, 0))
    return pl.pallas_call(
        _finish_body,
        grid=(nb,),
        in_specs=[pl.BlockSpec((BLK, D), lambda i: (i, 0))] + [col_spec] * 5,
        out_specs=(
            pl.BlockSpec((BLK, D), lambda i: (i, 0)),
            pl.BlockSpec((BLK, 4), lambda i: (i, 0)),
        ),
        out_shape=(
            jax.ShapeDtypeStruct((N, D), jnp.float32),
            jax.ShapeDtypeStruct((N, 4), jnp.int32),
        ),
    )(feats_sum, *cols)


def kernel(F, C, W, b):
    offsets, new_coords, h = _head(F, C, W, b)
    S, P = lax.sort_key_val(h, lax.iota(jnp.int32, N))
    r = _ranks(S)

    bnd = jnp.searchsorted(
        r, jnp.arange(NCHUNK, dtype=jnp.int32) * SPROWS, side="left"
    ).astype(jnp.int32)
    blo1 = jnp.zeros((160,), jnp.int32).at[:NCHUNK].set(bnd & ~7)
    bhi_core = jnp.concatenate(
        [bnd[1:], jnp.full((1,), N, jnp.int32)])
    bhi1 = jnp.zeros((160,), jnp.int32).at[:NCHUNK].set((bhi_core + 7) & ~7)
    blo = jnp.broadcast_to(blo1[:, None], (160, 16))
    bhi = jnp.broadcast_to(bhi1[:, None], (160, 16))
    pad_tgt = N + (jnp.arange(144, dtype=jnp.int32) % 64)
    P_pad = jnp.concatenate([P, pad_tgt])
    r_pad = jnp.concatenate([r, jnp.full((144,), 2 * N, jnp.int32)])
    inv = jnp.zeros((N,), jnp.int32).at[P].add(r)
    cols = [jnp.zeros((N,), jnp.float32).at[inv].add(
                new_coords[:, k].astype(jnp.float32))[:, None]
            for k in range(4)]
    cols.append(jnp.zeros((N,), jnp.float32).at[inv].add(1.0)[:, None])

    feats_sum = _pool_sc(F, P_pad, r_pad, blo, bhi)
    out_feats, out_coords = _finish(feats_sum, cols)
    return (offsets, out_coords, out_feats, inv)


# speedup vs baseline: 2.7004x; 2.7004x over previous
import functools

import jax
import jax.numpy as jnp
from jax import lax
from jax.experimental import pallas as pl
from jax.experimental.pallas import tpu as pltpu
from jax.experimental.pallas import tpu_sc as plsc

N = 320000
D = 128
BLK = 2000

SPROWS = 6144
NCHUNK = 54
STAGE = SPROWS + 16
W = 128
CPC = NCHUNK // 2
TS = SPROWS // 16


def _head_body(c_ref, f_ref, w_ref, b_ref, off_ref, nc_ref, h_ref, n8_ref):
    f = f_ref[...]
    w = w_ref[...]
    off = jnp.dot(f, w, preferred_element_type=jnp.float32) + b_ref[0, :]
    off_ref[...] = off
    ci = c_ref[...]
    oi = off.astype(jnp.int32)
    nc = ci + jnp.concatenate(
        [jnp.zeros((ci.shape[0], 1), jnp.int32), oi], axis=1)
    nc_ref[...] = nc
    n8_ref[...] = jnp.concatenate(
        [nc.astype(jnp.float32),
         jnp.ones((BLK, 1), jnp.float32),
         jnp.zeros((BLK, 3), jnp.float32)], axis=1)
    c = nc + 1024
    h = ((c[:, 0] * 4096 + c[:, 1]) * 4096 + c[:, 2]) * 4096 + c[:, 3]
    h_ref[...] = h.reshape(1, 1, BLK)


def _head(F, C, W, b):
    nb = N // BLK
    grid = (nb,)
    out_shapes = (
        jax.ShapeDtypeStruct((N, 3), jnp.float32),
        jax.ShapeDtypeStruct((N, 4), jnp.int32),
        jax.ShapeDtypeStruct((nb, 1, BLK), jnp.int32),
        jax.ShapeDtypeStruct((N, 8), jnp.float32),
    )
    off, nc, h, n8 = pl.pallas_call(
        _head_body,
        grid=grid,
        in_specs=[
            pl.BlockSpec((BLK, 4), lambda i: (i, 0)),
            pl.BlockSpec((BLK, D), lambda i: (i, 0)),
            pl.BlockSpec((D, 3), lambda i: (0, 0)),
            pl.BlockSpec((1, 3), lambda i: (0, 0)),
        ],
        out_specs=(
            pl.BlockSpec((BLK, 3), lambda i: (i, 0)),
            pl.BlockSpec((BLK, 4), lambda i: (i, 0)),
            pl.BlockSpec((1, 1, BLK), lambda i: (i, 0, 0)),
            pl.BlockSpec((BLK, 8), lambda i: (i, 0)),
        ),
        out_shape=out_shapes,
    )(C, F, W, b.reshape(1, 3))
    return off, nc, h.reshape(-1), n8


RBLK = 8000

def _rank_body(s_ref, r_ref, prev_ref, cum_ref):
    i = pl.program_id(0)

    @pl.when(i == 0)
    def _init():
        prev_ref[0] = s_ref[0, 0, 0] + 1
        cum_ref[0] = 0

    s = s_ref[0, 0, :]
    s_shift = jnp.concatenate([jnp.full((1,), prev_ref[0], jnp.int32), s[:-1]])
    flag = (s != s_shift).astype(jnp.int32)
    x = flag
    d = 1
    while d < RBLK:
        x = x + jnp.concatenate([jnp.zeros((d,), jnp.int32), x[:-d]])
        d *= 2
    r = x + cum_ref[0] - 1
    r_ref[0, 0, :] = r
    cum_ref[0] = r[-1] + 1
    prev_ref[0] = s[-1]


def _ranks(S):
    nb = N // RBLK
    r = pl.pallas_call(
        _rank_body,
        grid=(nb,),
        in_specs=[pl.BlockSpec((1, 1, RBLK), lambda i: (i, 0, 0))],
        out_specs=pl.BlockSpec((1, 1, RBLK), lambda i: (i, 0, 0)),
        out_shape=jax.ShapeDtypeStruct((nb, 1, RBLK), jnp.int32),
        scratch_shapes=[pltpu.SMEM((1,), jnp.int32), pltpu.SMEM((1,), jnp.int32)],
    )(S.reshape(nb, 1, RBLK))
    return r.reshape(-1)


def _pool_body(f_hbm, p_hbm, r_hbm, blo_hbm, bhi_hbm, z128_hbm,
               feats_out,
               blo_v, bhi_v, idx_v, pidx_v, rv_v, seg_v,
               rows_v, feats_st, sem1):
    core = lax.axis_index("c")
    sub = lax.axis_index("s")
    iota = lax.iota(jnp.int32, 16)

    pltpu.sync_copy(blo_hbm, blo_v)
    pltpu.sync_copy(bhi_hbm, bhi_v)
    zb0 = sub * TS
    pltpu.sync_copy(z128_hbm.at[pl.ds(zb0, TS)], feats_st.at[pl.ds(zb0, TS)])
    plsc.subcore_barrier()

    @pl.loop(0, CPC)
    def chunk_body(i):
        c = 2 * i + core
        lo_c = blo_v[c, :][0]
        hi_c = bhi_v[c, :][0]
        base_seg = c * SPROWS
        length = hi_c - lo_c
        lo_t = lo_c + (((length * sub) // 16) & ~7)
        hi_t = lo_c + (((length * (sub + 1)) // 16) & ~7)
        zbase = sub * TS

        nw = (hi_t - lo_t + (W - 1)) // W

        @pl.loop(0, nw)
        def win_body(w):
            j0 = pl.multiple_of(lo_t + w * W, 8)
            pltpu.sync_copy(p_hbm.at[pl.ds(j0, W)], idx_v)
            pltpu.sync_copy(r_hbm.at[pl.ds(j0, W)], rv_v)
            for k in range(W // 16):
                jvec = j0 + k * 16 + iota
                rv = rv_v[pl.ds(k * 16, 16)]
                seg = rv - base_seg
                ok = (jvec < hi_t) & (seg >= 0) & (seg < SPROWS)
                seg_v[pl.ds(k * 16, 16)] = jnp.where(ok, seg, SPROWS + iota)
                pv = idx_v[pl.ds(k * 16, 16)]
                pidx_v[pl.ds(k * 16, 16)] = jnp.minimum(pv, N - 1)
            pltpu.async_copy(f_hbm.at[pidx_v], rows_v, sem1).wait()
            pltpu.sync_copy(rows_v, feats_st.at[seg_v], add=True)

        plsc.subcore_barrier()

        obase = base_seg + zbase
        pltpu.sync_copy(feats_st.at[pl.ds(zbase, TS)],
                        feats_out.at[pl.ds(obase, TS)])
        pltpu.sync_copy(z128_hbm.at[pl.ds(zbase, TS)], feats_st.at[pl.ds(zbase, TS)])
        plsc.subcore_barrier()


def _pool_sc(F, P_pad, r_pad, blo, bhi):
    mesh = plsc.VectorSubcoreMesh(core_axis_name="c", subcore_axis_name="s")
    pool = pl.kernel(
        _pool_body,
        mesh=mesh,
        out_type=[
            jax.ShapeDtypeStruct((NCHUNK * SPROWS, D), jnp.float32),
        ],
        scratch_types=[
            pltpu.VMEM((160, 16), jnp.int32),
            pltpu.VMEM((160, 16), jnp.int32),
            pltpu.VMEM((W,), jnp.int32),
            pltpu.VMEM((W,), jnp.int32),
            pltpu.VMEM((W,), jnp.int32),
            pltpu.VMEM((W,), jnp.int32),
            pltpu.VMEM((W, D), jnp.float32),
            pltpu.VMEM_SHARED((STAGE, D), jnp.float32),
            pltpu.SemaphoreType.DMA,
        ],
    )
    z128 = jnp.zeros((SPROWS, D), jnp.float32)
    return pool(F, P_pad, r_pad, blo, bhi, z128)[0]


def _finish_body(fs_ref, cs_ref, feats_ref, coords_ref):
    cs = cs_ref[...]
    inv_c = 1.0 / jnp.maximum(cs[:, 4:5], 1.0)
    feats_ref[...] = fs_ref[...] * inv_c
    coords_ref[...] = (cs[:, :4] * inv_c).astype(jnp.int32)


def _finish(feats_sum, csum8):
    nb = N // BLK
    return pl.pallas_call(
        _finish_body,
        grid=(nb,),
        in_specs=[
            pl.BlockSpec((BLK, D), lambda i: (i, 0)),
            pl.BlockSpec((BLK, 8), lambda i: (i, 0)),
        ],
        out_specs=(
            pl.BlockSpec((BLK, D), lambda i: (i, 0)),
            pl.BlockSpec((BLK, 4), lambda i: (i, 0)),
        ),
        out_shape=(
            jax.ShapeDtypeStruct((N, D), jnp.float32),
            jax.ShapeDtypeStruct((N, 4), jnp.int32),
        ),
    )(feats_sum, csum8)


def kernel(F, C, W, b):
    offsets, new_coords, h, n8 = _head(F, C, W, b)
    S, P = lax.sort_key_val(h, lax.iota(jnp.int32, N))
    r = _ranks(S)

    bnd = jnp.searchsorted(
        r, jnp.arange(NCHUNK, dtype=jnp.int32) * SPROWS, side="left"
    ).astype(jnp.int32)
    blo1 = jnp.zeros((160,), jnp.int32).at[:NCHUNK].set(bnd & ~7)
    bhi_core = jnp.concatenate(
        [bnd[1:], jnp.full((1,), N, jnp.int32)])
    bhi1 = jnp.zeros((160,), jnp.int32).at[:NCHUNK].set((bhi_core + 7) & ~7)
    blo = jnp.broadcast_to(blo1[:, None], (160, 16))
    bhi = jnp.broadcast_to(bhi1[:, None], (160, 16))
    pad_tgt = N + (jnp.arange(144, dtype=jnp.int32) % 64)
    P_pad = jnp.concatenate([P, pad_tgt])
    r_pad = jnp.concatenate([r, jnp.full((144,), 2 * N, jnp.int32)])
    inv = jnp.zeros((N,), jnp.int32).at[P].add(
        r, unique_indices=True, mode="promise_in_bounds")
    csum8 = jnp.zeros((N, 8), jnp.float32).at[inv].add(
        n8, mode="promise_in_bounds")

    feats_sum = _pool_sc(F, P_pad, r_pad, blo, bhi)
    out_feats, out_coords = _finish(feats_sum, csum8)
    return (offsets, out_coords, out_feats, inv)


# speedup vs baseline: 2.8348x; 1.0498x over previous
import functools

import jax
import jax.numpy as jnp
from jax import lax
from jax.experimental import pallas as pl
from jax.experimental.pallas import tpu as pltpu
from jax.experimental.pallas import tpu_sc as plsc

N = 320000
D = 128
BLK = 2000

SPROWS = 6144
NCHUNK = 54
STAGE = SPROWS + 16
W = 128
CPC = NCHUNK // 2
TS = SPROWS // 16


def _head_body(c_ref, f_ref, w_ref, b_ref, off_ref, nc_ref, h_ref, n8_ref):
    f = f_ref[...]
    w = w_ref[...]
    off = jnp.dot(f, w, preferred_element_type=jnp.float32) + b_ref[0, :]
    off_ref[...] = off
    ci = c_ref[...]
    oi = off.astype(jnp.int32)
    nc = ci + jnp.concatenate(
        [jnp.zeros((ci.shape[0], 1), jnp.int32), oi], axis=1)
    nc_ref[...] = nc
    n8_ref[...] = jnp.concatenate(
        [nc.astype(jnp.float32),
         jnp.ones((BLK, 1), jnp.float32),
         jnp.zeros((BLK, 3), jnp.float32)], axis=1)
    c = nc + 1024
    h = ((c[:, 0] * 4096 + c[:, 1]) * 4096 + c[:, 2]) * 4096 + c[:, 3]
    h_ref[...] = h.reshape(1, 1, BLK)


def _head(F, C, W, b):
    nb = N // BLK
    grid = (nb,)
    out_shapes = (
        jax.ShapeDtypeStruct((N, 3), jnp.float32),
        jax.ShapeDtypeStruct((N, 4), jnp.int32),
        jax.ShapeDtypeStruct((nb, 1, BLK), jnp.int32),
        jax.ShapeDtypeStruct((N, 8), jnp.float32),
    )
    off, nc, h, n8 = pl.pallas_call(
        _head_body,
        grid=grid,
        in_specs=[
            pl.BlockSpec((BLK, 4), lambda i: (i, 0)),
            pl.BlockSpec((BLK, D), lambda i: (i, 0)),
            pl.BlockSpec((D, 3), lambda i: (0, 0)),
            pl.BlockSpec((1, 3), lambda i: (0, 0)),
        ],
        out_specs=(
            pl.BlockSpec((BLK, 3), lambda i: (i, 0)),
            pl.BlockSpec((BLK, 4), lambda i: (i, 0)),
            pl.BlockSpec((1, 1, BLK), lambda i: (i, 0, 0)),
            pl.BlockSpec((BLK, 8), lambda i: (i, 0)),
        ),
        out_shape=out_shapes,
    )(C, F, W, b.reshape(1, 3))
    return off, nc, h.reshape(-1), n8


RBLK = 8000

def _rank_body(s_ref, r_ref, prev_ref, cum_ref):
    i = pl.program_id(0)

    @pl.when(i == 0)
    def _init():
        prev_ref[0] = s_ref[0, 0, 0] + 1
        cum_ref[0] = 0

    s = s_ref[0, 0, :]
    s_shift = jnp.concatenate([jnp.full((1,), prev_ref[0], jnp.int32), s[:-1]])
    flag = (s != s_shift).astype(jnp.int32)
    x = flag
    d = 1
    while d < RBLK:
        x = x + jnp.concatenate([jnp.zeros((d,), jnp.int32), x[:-d]])
        d *= 2
    r = x + cum_ref[0] - 1
    r_ref[0, 0, :] = r
    cum_ref[0] = r[-1] + 1
    prev_ref[0] = s[-1]


def _ranks(S):
    nb = N // RBLK
    r = pl.pallas_call(
        _rank_body,
        grid=(nb,),
        in_specs=[pl.BlockSpec((1, 1, RBLK), lambda i: (i, 0, 0))],
        out_specs=pl.BlockSpec((1, 1, RBLK), lambda i: (i, 0, 0)),
        out_shape=jax.ShapeDtypeStruct((nb, 1, RBLK), jnp.int32),
        scratch_shapes=[pltpu.SMEM((1,), jnp.int32), pltpu.SMEM((1,), jnp.int32)],
    )(S.reshape(nb, 1, RBLK))
    return r.reshape(-1)


def _pool_body(f_hbm, p_hbm, r_hbm, blo_hbm, bhi_hbm, z128_hbm,
               feats_out,
               blo_v, bhi_v, idx_v, pidx_v, rv_v, seg_v,
               idx2_v, pidx2_v, rv2_v, seg2_v,
               rows_v, rows2_v, feats_st, sem1, sem2, sem_ld1, sem_ld2):
    core = lax.axis_index("c")
    sub = lax.axis_index("s")
    iota = lax.iota(jnp.int32, 16)

    pltpu.sync_copy(blo_hbm, blo_v)
    pltpu.sync_copy(bhi_hbm, bhi_v)
    zb0 = sub * TS
    pltpu.sync_copy(z128_hbm.at[pl.ds(zb0, TS)], feats_st.at[pl.ds(zb0, TS)])
    plsc.subcore_barrier()

    @pl.loop(0, CPC)
    def chunk_body(i):
        c = 2 * i + core
        lo_c = blo_v[c, :][0]
        hi_c = bhi_v[c, :][0]
        base_seg = c * SPROWS
        length = hi_c - lo_c
        lo_t = lo_c + (((length * sub) // 16) & ~7)
        hi_t = lo_c + (((length * (sub + 1)) // 16) & ~7)
        zbase = sub * TS

        nw = (hi_t - lo_t + (W - 1)) // W

        def _stage(w, idx_b, pidx_b, rv_b, seg_b, rows_b, sem_b):
            j0 = pl.multiple_of(lo_t + w * W, 8)
            ca = pltpu.async_copy(p_hbm.at[pl.ds(j0, W)], idx_b, sem_ld1)
            cb = pltpu.async_copy(r_hbm.at[pl.ds(j0, W)], rv_b, sem_ld2)
            ca.wait()
            cb.wait()
            for k in range(W // 16):
                jvec = j0 + k * 16 + iota
                rv = rv_b[pl.ds(k * 16, 16)]
                seg = rv - base_seg
                ok = (jvec < hi_t) & (seg >= 0) & (seg < SPROWS)
                seg_b[pl.ds(k * 16, 16)] = jnp.where(ok, seg, SPROWS + iota)
                pv = idx_b[pl.ds(k * 16, 16)]
                pidx_b[pl.ds(k * 16, 16)] = jnp.minimum(pv, N - 1)
            pltpu.async_copy(f_hbm.at[pidx_b], rows_b, sem_b)

        def _drain(rows_b, sem_b):
            pltpu.make_async_copy(f_hbm.at[pl.ds(0, W)], rows_b, sem_b).wait()

        @pl.when(nw > 0)
        def _prologue():
            _stage(0, idx_v, pidx_v, rv_v, seg_v, rows_v, sem1)

        @pl.loop(0, nw)
        def win_body(w):
            nxt = w + 1

            @pl.when((nxt < nw) & (nxt % 2 == 1))
            def _sb():
                _stage(nxt, idx2_v, pidx2_v, rv2_v, seg2_v, rows2_v, sem2)

            @pl.when((nxt < nw) & (nxt % 2 == 0))
            def _sa():
                _stage(nxt, idx_v, pidx_v, rv_v, seg_v, rows_v, sem1)

            @pl.when(w % 2 == 0)
            def _ca():
                _drain(rows_v, sem1)
                pltpu.sync_copy(rows_v, feats_st.at[seg_v], add=True)

            @pl.when(w % 2 == 1)
            def _cb():
                _drain(rows2_v, sem2)
                pltpu.sync_copy(rows2_v, feats_st.at[seg2_v], add=True)

        plsc.subcore_barrier()

        obase = base_seg + zbase
        pltpu.sync_copy(feats_st.at[pl.ds(zbase, TS)],
                        feats_out.at[pl.ds(obase, TS)])
        pltpu.sync_copy(z128_hbm.at[pl.ds(zbase, TS)], feats_st.at[pl.ds(zbase, TS)])
        plsc.subcore_barrier()


def _pool_sc(F, P_pad, r_pad, blo, bhi):
    mesh = plsc.VectorSubcoreMesh(core_axis_name="c", subcore_axis_name="s")
    pool = pl.kernel(
        _pool_body,
        mesh=mesh,
        out_type=[
            jax.ShapeDtypeStruct((NCHUNK * SPROWS, D), jnp.float32),
        ],
        scratch_types=[
            pltpu.VMEM((160, 16), jnp.int32),
            pltpu.VMEM((160, 16), jnp.int32),
            pltpu.VMEM((W,), jnp.int32),
            pltpu.VMEM((W,), jnp.int32),
            pltpu.VMEM((W,), jnp.int32),
            pltpu.VMEM((W,), jnp.int32),
            pltpu.VMEM((W,), jnp.int32),
            pltpu.VMEM((W,), jnp.int32),
            pltpu.VMEM((W,), jnp.int32),
            pltpu.VMEM((W,), jnp.int32),
            pltpu.VMEM((W, D), jnp.float32),
            pltpu.VMEM((W, D), jnp.float32),
            pltpu.VMEM_SHARED((STAGE, D), jnp.float32),
            pltpu.SemaphoreType.DMA,
            pltpu.SemaphoreType.DMA,
            pltpu.SemaphoreType.DMA,
            pltpu.SemaphoreType.DMA,
        ],
    )
    z128 = jnp.zeros((SPROWS, D), jnp.float32)
    return pool(F, P_pad, r_pad, blo, bhi, z128)[0]


def _finish_body(fs_ref, cs_ref, feats_ref, coords_ref):
    cs = cs_ref[...]
    inv_c = 1.0 / jnp.maximum(cs[:, 4:5], 1.0)
    feats_ref[...] = fs_ref[...] * inv_c
    coords_ref[...] = (cs[:, :4] * inv_c).astype(jnp.int32)


def _finish(feats_sum, csum8):
    nb = N // BLK
    return pl.pallas_call(
        _finish_body,
        grid=(nb,),
        in_specs=[
            pl.BlockSpec((BLK, D), lambda i: (i, 0)),
            pl.BlockSpec((BLK, 8), lambda i: (i, 0)),
        ],
        out_specs=(
            pl.BlockSpec((BLK, D), lambda i: (i, 0)),
            pl.BlockSpec((BLK, 4), lambda i: (i, 0)),
        ),
        out_shape=(
            jax.ShapeDtypeStruct((N, D), jnp.float32),
            jax.ShapeDtypeStruct((N, 4), jnp.int32),
        ),
    )(feats_sum, csum8)


def kernel(F, C, W, b):
    offsets, new_coords, h, n8 = _head(F, C, W, b)
    S, P = lax.sort_key_val(h, lax.iota(jnp.int32, N))
    r = _ranks(S)

    bnd = jnp.searchsorted(
        r, jnp.arange(NCHUNK, dtype=jnp.int32) * SPROWS, side="left"
    ).astype(jnp.int32)
    blo1 = jnp.zeros((160,), jnp.int32).at[:NCHUNK].set(bnd & ~7)
    bhi_core = jnp.concatenate(
        [bnd[1:], jnp.full((1,), N, jnp.int32)])
    bhi1 = jnp.zeros((160,), jnp.int32).at[:NCHUNK].set((bhi_core + 7) & ~7)
    blo = jnp.broadcast_to(blo1[:, None], (160, 16))
    bhi = jnp.broadcast_to(bhi1[:, None], (160, 16))
    pad_tgt = N + (jnp.arange(144, dtype=jnp.int32) % 64)
    P_pad = jnp.concatenate([P, pad_tgt])
    r_pad = jnp.concatenate([r, jnp.full((144,), 2 * N, jnp.int32)])
    inv = jnp.zeros((N,), jnp.int32).at[P].add(
        r, unique_indices=True, mode="promise_in_bounds")
    csum8 = jnp.zeros((N, 8), jnp.float32).at[inv].add(
        n8, mode="promise_in_bounds")

    feats_sum = _pool_sc(F, P_pad, r_pad, blo, bhi)
    out_feats, out_coords = _finish(feats_sum, csum8)
    return (offsets, out_coords, out_feats, inv)
